# Initial kernel scaffold; baseline (speedup 1.0000x reference)
import jax
import jax.numpy as jnp
from jax.experimental import pallas as pl


def kernel(H, edge_index, norm_factor, W_node, b_node, W_update, b_update):
    raise NotImplementedError("write your pallas kernel here")


# speedup vs baseline: 5.5316x; 5.5316x over previous
import functools

import jax
import jax.numpy as jnp
from jax import lax
from jax.experimental import pallas as pl
from jax.experimental.pallas import tpu as pltpu
from jax.experimental.pallas import tpu_sc as plsc

N = 10000
E = 320000
D = 128

NC = 2
NS = 16
L = 16
NW = NC * NS
EPW = E // NW
C = 80
NCH = EPW // C
NPAD = 10240
RPT = NPAD // NS
NB = 5
CPB = NCH // NB
EPB = CPB * C


def _enc_body(x_ref, w_ref, b_ref, o_ref):
    o_ref[...] = (
        jnp.dot(x_ref[...], w_ref[...], preferred_element_type=jnp.float32)
        + b_ref[...]
    )


def _update_body(p_ref, h_ref, w_ref, b_ref, o_ref):
    agg = p_ref[0] + p_ref[1] + h_ref[...]
    o_ref[...] = jnp.maximum(
        jnp.dot(agg, w_ref[...], preferred_element_type=jnp.float32) + b_ref[...],
        0.0,
    )


def _sc_body(henc, u3, v3, n3, out, u_v, v_v, n_v, rows_v, agg, sem):
    c = lax.axis_index("c")
    s = lax.axis_index("s")
    wid = c * NS + s

    def zero_row(i):
        for d in range(D // L):
            rows_v[i, pl.ds(d * L, L)] = jnp.zeros((L,), jnp.float32)

    pl.loop(0, C)(zero_row)

    for k in range(RPT // C):
        pltpu.sync_copy(rows_v, agg.at[pl.ds(s * RPT + k * C, C)])
    plsc.subcore_barrier()

    for b in range(NB):
        pltpu.sync_copy(u3.at[wid].at[b], u_v)
        pltpu.sync_copy(v3.at[wid].at[b], v_v)
        pltpu.sync_copy(n3.at[wid].at[b], n_v)

        def do_chunk(j):
            pltpu.async_copy(henc.at[u_v.at[j]], rows_v, sem).wait()

            def scale_edge(e):
                j16 = jnp.full((L,), j, jnp.int32)
                e16 = jnp.full((L,), e, jnp.int32)
                nv = plsc.load_gather(n_v, [j16, e16])
                for d in range(D // L):
                    sl = pl.ds(d * L, L)
                    rows_v[e, sl] = rows_v[e, sl] * nv

            pl.loop(0, C)(scale_edge)

            pltpu.sync_copy(rows_v, agg.at[v_v.at[j]], add=True)

        pl.loop(0, CPB)(do_chunk)

    plsc.subcore_barrier()
    pltpu.sync_copy(
        agg.at[pl.ds(s * RPT, RPT)], out.at[c].at[pl.ds(s * RPT, RPT)]
    )


_sc_call = pl.kernel(
    _sc_body,
    out_type=jax.ShapeDtypeStruct((NC, NPAD, D), jnp.float32),
    mesh=plsc.VectorSubcoreMesh(core_axis_name="c", subcore_axis_name="s"),
    scratch_types=[
        pltpu.VMEM((CPB, C), jnp.int32),
        pltpu.VMEM((CPB, C), jnp.int32),
        pltpu.VMEM((CPB, C), jnp.float32),
        pltpu.VMEM((C, D), jnp.float32),
        pltpu.VMEM_SHARED((NPAD, D), jnp.float32),
        pltpu.SemaphoreType.DMA,
    ],
    compiler_params=pltpu.CompilerParams(needs_layout_passes=False),
)


def kernel(H, edge_index, norm_factor, W_node, b_node, W_update, b_update):
    RB = 1000
    henc = pl.pallas_call(
        _enc_body,
        grid=(N // RB,),
        in_specs=[
            pl.BlockSpec((RB, D), lambda i: (i, 0)),
            pl.BlockSpec((D, D), lambda i: (0, 0)),
            pl.BlockSpec((1, D), lambda i: (0, 0)),
        ],
        out_specs=pl.BlockSpec((RB, D), lambda i: (i, 0)),
        out_shape=jax.ShapeDtypeStruct((N, D), jnp.float32),
    )(H, W_node, b_node.reshape(1, D))

    u3 = edge_index[0].astype(jnp.int32).reshape(NW, NB, CPB, C)
    v3 = edge_index[1].astype(jnp.int32).reshape(NW, NB, CPB, C)
    n3 = norm_factor.astype(jnp.float32).reshape(NW, NB, CPB, C)

    partials = _sc_call(henc, u3, v3, n3)

    out = pl.pallas_call(
        _update_body,
        grid=(N // RB,),
        in_specs=[
            pl.BlockSpec((NC, RB, D), lambda i: (0, i, 0)),
            pl.BlockSpec((RB, D), lambda i: (i, 0)),
            pl.BlockSpec((D, D), lambda i: (0, 0)),
            pl.BlockSpec((1, D), lambda i: (0, 0)),
        ],
        out_specs=pl.BlockSpec((RB, D), lambda i: (i, 0)),
        out_shape=jax.ShapeDtypeStruct((N, D), jnp.float32),
    )(partials, henc, W_update, b_update.reshape(1, D))
    return out


# speedup vs baseline: 9.7322x; 1.7594x over previous
import functools

import jax
import jax.numpy as jnp
from jax import lax
from jax.experimental import pallas as pl
from jax.experimental.pallas import tpu as pltpu
from jax.experimental.pallas import tpu_sc as plsc

N = 10000
E = 320000
D = 128

NC = 2
NS = 16
L = 16
NW = NC * NS
EPW = E // NW
C = 80
NCH = EPW // C
NPAD = 10112
RPT = NPAD // NS
NB = 5
CPB = NCH // NB
EPB = CPB * C
NBUF = 3


def _enc_body(x_ref, w_ref, b_ref, o_ref):
    o_ref[...] = (
        jnp.dot(x_ref[...], w_ref[...], preferred_element_type=jnp.float32)
        + b_ref[...]
    )


def _update_body(p_ref, h_ref, w_ref, b_ref, o_ref):
    agg = p_ref[0] + p_ref[1] + h_ref[...]
    o_ref[...] = jnp.maximum(
        jnp.dot(agg, w_ref[...], preferred_element_type=jnp.float32) + b_ref[...],
        0.0,
    )


def _sc_body(henc, u3, v3, n3, out, u_v, v_v, n_v, r0, r1, r2, agg,
             g0, g1, g2, s0, s1, s2):
    c = lax.axis_index("c")
    s = lax.axis_index("s")
    wid = c * NS + s
    rows = (r0, r1, r2)
    gsem = (g0, g1, g2)
    ssem = (s0, s1, s2)

    def zero_row(i):
        for d in range(D // L):
            r0[i, pl.ds(d * L, L)] = jnp.zeros((L,), jnp.float32)

    pl.loop(0, C)(zero_row)

    for k in range(RPT // C):
        pltpu.sync_copy(r0, agg.at[pl.ds(s * RPT + k * C, C)])
    rem = RPT - (RPT // C) * C
    if rem:
        pltpu.sync_copy(
            r0.at[pl.ds(0, rem)], agg.at[pl.ds(s * RPT + (RPT // C) * C, rem)]
        )
    plsc.subcore_barrier()

    def gather_start(j, x):
        pltpu.async_copy(henc.at[u_v.at[j]], rows[x], gsem[x])

    def gather_wait(j, x):
        pltpu.make_async_copy(henc.at[u_v.at[j]], rows[x], gsem[x]).wait()

    def scatter_start(j, x):
        pltpu.async_copy(rows[x], agg.at[v_v.at[j]], ssem[x], add=True)

    def scatter_wait(j, x):
        pltpu.make_async_copy(rows[x], agg.at[v_v.at[j]], ssem[x]).wait()

    def scale(j, x):
        buf = rows[x]

        def scale_edge(e):
            j16 = jnp.full((L,), j, jnp.int32)
            e16 = jnp.full((L,), e, jnp.int32)
            nv = plsc.load_gather(n_v, [j16, e16])
            for d in range(D // L):
                sl = pl.ds(d * L, L)
                buf[e, sl] = buf[e, sl] * nv

        pl.loop(0, C, unroll=4)(scale_edge)

    for b in range(NB):
        pltpu.sync_copy(u3.at[wid].at[b], u_v)
        pltpu.sync_copy(v3.at[wid].at[b], v_v)
        pltpu.sync_copy(n3.at[wid].at[b], n_v)

        gather_start(0, 0)
        gather_start(1, 1)
        gather_wait(0, 0)
        scale(0, 0)
        gather_start(2, 2)
        scatter_start(0, 0)

        def triple(t):
            for r in range(3):
                j = 3 * t + 1 + r
                x = (1 + r) % 3
                y = r
                gather_wait(j, x)
                scale(j, x)
                scatter_wait(j - 1, y)
                if r == 0:
                    gather_start(j + 2, y)
                else:
                    @pl.when(t < (CPB - 2) // 3 - 1 + 1)
                    def _():
                        gather_start(j + 2, y)
                scatter_start(j, x)

        pl.loop(0, (CPB - 1) // 3)(triple)
        scatter_wait(CPB - 1, (CPB - 1) % 3)

    plsc.subcore_barrier()
    pltpu.sync_copy(
        agg.at[pl.ds(s * RPT, RPT)], out.at[c].at[pl.ds(s * RPT, RPT)]
    )


_sc_call = pl.kernel(
    _sc_body,
    out_type=jax.ShapeDtypeStruct((NC, NPAD, D), jnp.float32),
    mesh=plsc.VectorSubcoreMesh(core_axis_name="c", subcore_axis_name="s"),
    scratch_types=[
        pltpu.VMEM((CPB, C), jnp.int32),
        pltpu.VMEM((CPB, C), jnp.int32),
        pltpu.VMEM((CPB, C), jnp.float32),
        pltpu.VMEM((C, D), jnp.float32),
        pltpu.VMEM((C, D), jnp.float32),
        pltpu.VMEM((C, D), jnp.float32),
        pltpu.VMEM_SHARED((NPAD, D), jnp.float32),
        pltpu.SemaphoreType.DMA,
        pltpu.SemaphoreType.DMA,
        pltpu.SemaphoreType.DMA,
        pltpu.SemaphoreType.DMA,
        pltpu.SemaphoreType.DMA,
        pltpu.SemaphoreType.DMA,
    ],
    compiler_params=pltpu.CompilerParams(needs_layout_passes=False),
)


def kernel(H, edge_index, norm_factor, W_node, b_node, W_update, b_update):
    RB = 1000
    henc = pl.pallas_call(
        _enc_body,
        grid=(N // RB,),
        in_specs=[
            pl.BlockSpec((RB, D), lambda i: (i, 0)),
            pl.BlockSpec((D, D), lambda i: (0, 0)),
            pl.BlockSpec((1, D), lambda i: (0, 0)),
        ],
        out_specs=pl.BlockSpec((RB, D), lambda i: (i, 0)),
        out_shape=jax.ShapeDtypeStruct((N, D), jnp.float32),
    )(H, W_node, b_node.reshape(1, D))

    u3 = edge_index[0].astype(jnp.int32).reshape(NW, NB, CPB, C)
    v3 = edge_index[1].astype(jnp.int32).reshape(NW, NB, CPB, C)
    n3 = norm_factor.astype(jnp.float32).reshape(NW, NB, CPB, C)

    partials = _sc_call(henc, u3, v3, n3)

    out = pl.pallas_call(
        _update_body,
        grid=(N // RB,),
        in_specs=[
            pl.BlockSpec((NC, RB, D), lambda i: (0, i, 0)),
            pl.BlockSpec((RB, D), lambda i: (i, 0)),
            pl.BlockSpec((D, D), lambda i: (0, 0)),
            pl.BlockSpec((1, D), lambda i: (0, 0)),
        ],
        out_specs=pl.BlockSpec((RB, D), lambda i: (i, 0)),
        out_shape=jax.ShapeDtypeStruct((N, D), jnp.float32),
    )(partials, henc, W_update, b_update.reshape(1, D))
    return out


# speedup vs baseline: 10.8173x; 1.1115x over previous
import functools

import numpy as np
import jax
import jax.numpy as jnp
from jax import lax
from jax.experimental import pallas as pl
from jax.experimental.pallas import tpu as pltpu
from jax.experimental.pallas import tpu_sc as plsc

N = 10000
E = 320000
D = 128

NC = 2
NS = 16
L = 16
NW = NC * NS
EPW = E // NW
C = 80
NCH = EPW // C
NPAD = 10112
RPT = NPAD // NS
NB = 5
CPB = NCH // NB
EPB = CPB * C
NBUF = 3
_GDN = jax.lax.GatherDimensionNumbers(
    offset_dims=(), collapsed_slice_dims=(0,), start_index_map=(0,)
)


def _splat_lane(vec, r):
    idx = jax.lax.full((L, 1), jnp.int32(r), jnp.int32)
    return jax.lax.gather(
        vec, idx, _GDN, (1,),
        mode=jax.lax.GatherScatterMode.PROMISE_IN_BOUNDS,
    )


def _enc_body(x_ref, w_ref, b_ref, o_ref):
    o_ref[...] = (
        jnp.dot(x_ref[...], w_ref[...], preferred_element_type=jnp.float32)
        + b_ref[...]
    )


def _update_body(p_ref, h_ref, w_ref, b_ref, o_ref):
    agg = p_ref[0] + p_ref[1] + h_ref[...]
    o_ref[...] = jnp.maximum(
        jnp.dot(agg, w_ref[...], preferred_element_type=jnp.float32) + b_ref[...],
        0.0,
    )


def _sc_body(henc, u3, v3, n3, out, u_v, v_v, n_v, r0, r1, r2, agg,
             g0, g1, g2, s0, s1, s2):
    c = lax.axis_index("c")
    s = lax.axis_index("s")
    wid = c * NS + s
    rows = (r0, r1, r2)
    gsem = (g0, g1, g2)
    ssem = (s0, s1, s2)

    def zero_row(i):
        for d in range(D // L):
            r0[i, pl.ds(d * L, L)] = jnp.zeros((L,), jnp.float32)

    pl.loop(0, C)(zero_row)

    for k in range(RPT // C):
        pltpu.sync_copy(r0, agg.at[pl.ds(s * RPT + k * C, C)])
    rem = RPT - (RPT // C) * C
    if rem:
        pltpu.sync_copy(
            r0.at[pl.ds(0, rem)], agg.at[pl.ds(s * RPT + (RPT // C) * C, rem)]
        )
    plsc.subcore_barrier()

    def gather_start(j, x):
        pltpu.async_copy(henc.at[u_v.at[j]], rows[x], gsem[x])

    def gather_wait(j, x):
        pltpu.make_async_copy(henc.at[u_v.at[j]], rows[x], gsem[x]).wait()

    def scatter_start(j, x):
        pltpu.async_copy(rows[x], agg.at[v_v.at[j]], ssem[x], add=True)

    def scatter_wait(j, x):
        pltpu.make_async_copy(rows[x], agg.at[v_v.at[j]], ssem[x]).wait()

    def scale(j, x):
        buf = rows[x]

        def group(g):
            nv16 = n_v[j, pl.ds(g * L, L)]
            for r in range(L):
                e = g * L + r
                nvb = _splat_lane(nv16, r)
                for d in range(D // L):
                    sl = pl.ds(d * L, L)
                    buf[e, sl] = buf[e, sl] * nvb

        pl.loop(0, C // L)(group)

    def block(b):
        pltpu.sync_copy(u3.at[wid].at[b], u_v)
        pltpu.sync_copy(v3.at[wid].at[b], v_v)
        pltpu.sync_copy(n3.at[wid].at[b], n_v)

        gather_start(0, 0)
        gather_start(1, 1)
        gather_wait(0, 0)
        scale(0, 0)
        gather_start(2, 2)
        scatter_start(0, 0)

        def triple(t):
            for r in range(3):
                j = 3 * t + 1 + r
                x = (1 + r) % 3
                y = r
                gather_wait(j, x)
                scale(j, x)
                scatter_wait(j - 1, y)
                if r == 0:
                    gather_start(j + 2, y)
                else:
                    @pl.when(t < (CPB - 1) // 3 - 1)
                    def _():
                        gather_start(j + 2, y)
                scatter_start(j, x)

        pl.loop(0, (CPB - 1) // 3)(triple)
        scatter_wait(CPB - 1, (CPB - 1) % 3)

    pl.loop(0, NB)(block)
    plsc.subcore_barrier()
    pltpu.sync_copy(
        agg.at[pl.ds(s * RPT, RPT)], out.at[c].at[pl.ds(s * RPT, RPT)]
    )


_sc_call = pl.kernel(
    _sc_body,
    out_type=jax.ShapeDtypeStruct((NC, NPAD, D), jnp.float32),
    mesh=plsc.VectorSubcoreMesh(core_axis_name="c", subcore_axis_name="s"),
    scratch_types=[
        pltpu.VMEM((CPB, C), jnp.int32),
        pltpu.VMEM((CPB, C), jnp.int32),
        pltpu.VMEM((CPB, C), jnp.float32),
        pltpu.VMEM((C, D), jnp.float32),
        pltpu.VMEM((C, D), jnp.float32),
        pltpu.VMEM((C, D), jnp.float32),
        pltpu.VMEM_SHARED((NPAD, D), jnp.float32),
        pltpu.SemaphoreType.DMA,
        pltpu.SemaphoreType.DMA,
        pltpu.SemaphoreType.DMA,
        pltpu.SemaphoreType.DMA,
        pltpu.SemaphoreType.DMA,
        pltpu.SemaphoreType.DMA,
    ],
    compiler_params=pltpu.CompilerParams(needs_layout_passes=False),
)


def kernel(H, edge_index, norm_factor, W_node, b_node, W_update, b_update):
    RB = 1000
    henc = pl.pallas_call(
        _enc_body,
        grid=(N // RB,),
        in_specs=[
            pl.BlockSpec((RB, D), lambda i: (i, 0)),
            pl.BlockSpec((D, D), lambda i: (0, 0)),
            pl.BlockSpec((1, D), lambda i: (0, 0)),
        ],
        out_specs=pl.BlockSpec((RB, D), lambda i: (i, 0)),
        out_shape=jax.ShapeDtypeStruct((N, D), jnp.float32),
    )(H, W_node, b_node.reshape(1, D))

    u3 = edge_index[0].astype(jnp.int32).reshape(NW, NB, CPB, C)
    v3 = edge_index[1].astype(jnp.int32).reshape(NW, NB, CPB, C)
    n3 = norm_factor.astype(jnp.float32).reshape(NW, NB, CPB, C)

    partials = _sc_call(henc, u3, v3, n3)

    out = pl.pallas_call(
        _update_body,
        grid=(N // RB,),
        in_specs=[
            pl.BlockSpec((NC, RB, D), lambda i: (0, i, 0)),
            pl.BlockSpec((RB, D), lambda i: (i, 0)),
            pl.BlockSpec((D, D), lambda i: (0, 0)),
            pl.BlockSpec((1, D), lambda i: (0, 0)),
        ],
        out_specs=pl.BlockSpec((RB, D), lambda i: (i, 0)),
        out_shape=jax.ShapeDtypeStruct((N, D), jnp.float32),
    )(partials, henc, W_update, b_update.reshape(1, D))
    return out


# speedup vs baseline: 10.8228x; 1.0005x over previous
import functools

import numpy as np
import jax
import jax.numpy as jnp
from jax import lax
from jax.experimental import pallas as pl
from jax.experimental.pallas import tpu as pltpu
from jax.experimental.pallas import tpu_sc as plsc

N = 10000
E = 320000
D = 128

NC = 2
NS = 16
L = 16
NW = NC * NS
EPW = E // NW
C = 80
NCH = EPW // C
NPAD = 10112
RPT = NPAD // NS
NB = 5
CPB = NCH // NB
EPB = CPB * C
NBUF = 3
_GDN = jax.lax.GatherDimensionNumbers(
    offset_dims=(), collapsed_slice_dims=(0,), start_index_map=(0,)
)


def _splat_lane(vec, r):
    idx = jax.lax.full((L, 1), jnp.int32(r), jnp.int32)
    return jax.lax.gather(
        vec, idx, _GDN, (1,),
        mode=jax.lax.GatherScatterMode.PROMISE_IN_BOUNDS,
    )


def _enc_body(x_ref, w_ref, b_ref, o_ref):
    o_ref[...] = (
        jnp.dot(x_ref[...], w_ref[...], preferred_element_type=jnp.float32)
        + b_ref[...]
    )


def _update_body(p_ref, h_ref, w_ref, b_ref, o_ref):
    agg = p_ref[0] + p_ref[1] + h_ref[...]
    o_ref[...] = jnp.maximum(
        jnp.dot(agg, w_ref[...], preferred_element_type=jnp.float32) + b_ref[...],
        0.0,
    )


def _sc_body(henc, u3, v3, n3, out, u_v, v_v, n_v, r0, r1, r2, agg,
             g0, g1, g2, s0, s1, s2):
    c = lax.axis_index("c")
    s = lax.axis_index("s")
    wid = c * NS + s
    rows = (r0, r1, r2)
    gsem = (g0, g1, g2)
    ssem = (s0, s1, s2)

    def zero_row(i):
        for d in range(D // L):
            r0[i, pl.ds(d * L, L)] = jnp.zeros((L,), jnp.float32)

    pl.loop(0, C)(zero_row)

    for k in range(RPT // C):
        pltpu.sync_copy(r0, agg.at[pl.ds(s * RPT + k * C, C)])
    rem = RPT - (RPT // C) * C
    if rem:
        pltpu.sync_copy(
            r0.at[pl.ds(0, rem)], agg.at[pl.ds(s * RPT + (RPT // C) * C, rem)]
        )
    plsc.subcore_barrier()

    def gather_start(j, p):
        pltpu.async_copy(henc.at[u_v.at[j]], rows[p], gsem[p])

    def gather_wait(j, p):
        pltpu.make_async_copy(henc.at[u_v.at[j]], rows[p], gsem[p]).wait()

    def scatter_start(j, p):
        pltpu.async_copy(rows[p], agg.at[v_v.at[j]], ssem[p], add=True)

    def scatter_wait(j, p):
        pltpu.make_async_copy(rows[p], agg.at[v_v.at[j]], ssem[p]).wait()

    def scale(j, p):
        buf = rows[p]

        def group(g):
            nv16 = n_v[j, pl.ds(g * L, L)]
            for r in range(L):
                e = g * L + r
                nvb = _splat_lane(nv16, r)
                for d in range(D // L):
                    sl = pl.ds(d * L, L)
                    buf[e, sl] = buf[e, sl] * nvb

        pl.loop(0, C // L)(group)

    def block(b):
        pltpu.sync_copy(u3.at[wid].at[b], u_v)
        pltpu.sync_copy(v3.at[wid].at[b], v_v)
        pltpu.sync_copy(n3.at[wid].at[b], n_v)

        gather_start(0, 0)
        gather_start(1, 1)
        gather_wait(0, 0)
        scale(0, 0)
        gather_start(2, 2)
        scatter_start(0, 0)

        def triple(t):
            for r in range(3):
                j = 3 * t + 1 + r
                x = (1 + r) % 3
                y = r
                gather_wait(j, x)
                scale(j, x)
                scatter_wait(j - 1, y)
                if r == 0:
                    gather_start(j + 2, y)
                else:
                    @pl.when(t < (CPB - 1) // 3 - 1)
                    def _():
                        gather_start(j + 2, y)
                scatter_start(j, x)

        pl.loop(0, (CPB - 1) // 3)(triple)
        scatter_wait(CPB - 1, (CPB - 1) % 3)

    pl.loop(0, NB)(block)
    plsc.subcore_barrier()
    pltpu.sync_copy(
        agg.at[pl.ds(s * RPT, RPT)], out.at[c].at[pl.ds(s * RPT, RPT)]
    )


_sc_call = pl.kernel(
    _sc_body,
    out_type=jax.ShapeDtypeStruct((NC, NPAD, D), jnp.float32),
    mesh=plsc.VectorSubcoreMesh(core_axis_name="c", subcore_axis_name="s"),
    scratch_types=[
        pltpu.VMEM((CPB, C), jnp.int32),
        pltpu.VMEM((CPB, C), jnp.int32),
        pltpu.VMEM((CPB, C), jnp.float32),
        pltpu.VMEM((C, D), jnp.float32),
        pltpu.VMEM((C, D), jnp.float32),
        pltpu.VMEM((C, D), jnp.float32),
        pltpu.VMEM_SHARED((NPAD, D), jnp.float32),
        pltpu.SemaphoreType.DMA,
        pltpu.SemaphoreType.DMA,
        pltpu.SemaphoreType.DMA,
        pltpu.SemaphoreType.DMA,
        pltpu.SemaphoreType.DMA,
        pltpu.SemaphoreType.DMA,
    ],
    compiler_params=pltpu.CompilerParams(needs_layout_passes=False),
)


def kernel(H, edge_index, norm_factor, W_node, b_node, W_update, b_update):
    RB = 1000
    henc = pl.pallas_call(
        _enc_body,
        grid=(N // RB,),
        in_specs=[
            pl.BlockSpec((RB, D), lambda i: (i, 0)),
            pl.BlockSpec((D, D), lambda i: (0, 0)),
            pl.BlockSpec((1, D), lambda i: (0, 0)),
        ],
        out_specs=pl.BlockSpec((RB, D), lambda i: (i, 0)),
        out_shape=jax.ShapeDtypeStruct((N, D), jnp.float32),
    )(H, W_node, b_node.reshape(1, D))

    u3 = edge_index[0].astype(jnp.int32).reshape(NW, NB, CPB, C)
    v3 = edge_index[1].astype(jnp.int32).reshape(NW, NB, CPB, C)
    n3 = norm_factor.astype(jnp.float32).reshape(NW, NB, CPB, C)

    partials = _sc_call(henc, u3, v3, n3)

    out = pl.pallas_call(
        _update_body,
        grid=(N // RB,),
        in_specs=[
            pl.BlockSpec((NC, RB, D), lambda i: (0, i, 0)),
            pl.BlockSpec((RB, D), lambda i: (i, 0)),
            pl.BlockSpec((D, D), lambda i: (0, 0)),
            pl.BlockSpec((1, D), lambda i: (0, 0)),
        ],
        out_specs=pl.BlockSpec((RB, D), lambda i: (i, 0)),
        out_shape=jax.ShapeDtypeStruct((N, D), jnp.float32),
    )(partials, henc, W_update, b_update.reshape(1, D))
    return out


# speedup vs baseline: 11.3328x; 1.0471x over previous
import functools

import numpy as np
import jax
import jax.numpy as jnp
from jax import lax
from jax.experimental import pallas as pl
from jax.experimental.pallas import tpu as pltpu
from jax.experimental.pallas import tpu_sc as plsc

N = 10000
E = 320000
D = 128

NC = 2
NS = 16
L = 16
NW = NC * NS
EPW = E // NW
C = 80
NCH = EPW // C
NPAD = 10112
RPT = NPAD // NS
NB = 5
CPB = NCH // NB
EPB = CPB * C
NBUF = 3
_GDN = jax.lax.GatherDimensionNumbers(
    offset_dims=(), collapsed_slice_dims=(0,), start_index_map=(0,)
)


def _splat_lane(vec, r):
    idx = jax.lax.full((L, 1), jnp.int32(r), jnp.int32)
    return jax.lax.gather(
        vec, idx, _GDN, (1,),
        mode=jax.lax.GatherScatterMode.PROMISE_IN_BOUNDS,
    )


def _enc_body(x_ref, w_ref, b_ref, o_ref):
    o_ref[...] = (
        jnp.dot(x_ref[...], w_ref[...], preferred_element_type=jnp.float32)
        + b_ref[...]
    )


def _update_body(p_ref, h_ref, w_ref, b_ref, o_ref):
    agg = p_ref[0] + p_ref[1] + h_ref[...]
    o_ref[...] = jnp.maximum(
        jnp.dot(agg, w_ref[...], preferred_element_type=jnp.float32) + b_ref[...],
        0.0,
    )


def _sc_body(henc, u3, v3, n3, out, u_v, v_v, n_v, r0, r1, r2, agg,
             g0, g1, g2, s0, s1, s2):
    c = lax.axis_index("c")
    s = lax.axis_index("s")
    wid = c * NS + s
    rows = (r0, r1, r2)
    gsem = (g0, g1, g2)
    ssem = (s0, s1, s2)

    def zero_row(i):
        for d in range(D // L):
            r0[i, pl.ds(d * L, L)] = jnp.zeros((L,), jnp.float32)

    pl.loop(0, C)(zero_row)

    nz = RPT // C
    rem = RPT - nz * C
    for k in range(nz):
        pltpu.async_copy(r0, agg.at[pl.ds(s * RPT + k * C, C)], gsem[k % 3])
    if rem:
        pltpu.async_copy(
            r0.at[pl.ds(0, rem)],
            agg.at[pl.ds(s * RPT + nz * C, rem)],
            gsem[nz % 3],
        )
    for k in range(nz):
        pltpu.make_async_copy(
            r0, agg.at[pl.ds(s * RPT + k * C, C)], gsem[k % 3]
        ).wait()
    if rem:
        pltpu.make_async_copy(
            r0.at[pl.ds(0, rem)],
            agg.at[pl.ds(s * RPT + nz * C, rem)],
            gsem[nz % 3],
        ).wait()
    plsc.subcore_barrier()

    def gather_start(j, p):
        pltpu.async_copy(henc.at[u_v.at[j]], rows[p], gsem[p])

    def gather_wait(j, p):
        pltpu.make_async_copy(henc.at[u_v.at[j]], rows[p], gsem[p]).wait()

    def scatter_start(j, p):
        pltpu.async_copy(rows[p], agg.at[v_v.at[j]], ssem[p], add=True)

    def scatter_wait(j, p):
        pltpu.make_async_copy(rows[p], agg.at[v_v.at[j]], ssem[p]).wait()

    def scale(j, p):
        buf = rows[p]

        def group(g):
            nv16 = n_v[j, pl.ds(g * L, L)]
            for r in range(L):
                e = g * L + r
                nvb = _splat_lane(nv16, r)
                for d in range(D // L):
                    sl = pl.ds(d * L, L)
                    buf[e, sl] = buf[e, sl] * nvb

        pl.loop(0, C // L)(group)

    def block(b):
        pltpu.async_copy(u3.at[wid].at[b], u_v, gsem[0])
        pltpu.async_copy(v3.at[wid].at[b], v_v, gsem[1])
        pltpu.async_copy(n3.at[wid].at[b], n_v, gsem[2])
        pltpu.make_async_copy(u3.at[wid].at[b], u_v, gsem[0]).wait()
        pltpu.make_async_copy(v3.at[wid].at[b], v_v, gsem[1]).wait()
        pltpu.make_async_copy(n3.at[wid].at[b], n_v, gsem[2]).wait()

        gather_start(0, 0)
        gather_start(1, 1)
        gather_wait(0, 0)
        scale(0, 0)
        gather_start(2, 2)
        scatter_start(0, 0)

        def triple(t):
            for r in range(3):
                j = 3 * t + 1 + r
                x = (1 + r) % 3
                y = r
                gather_wait(j, x)
                scale(j, x)
                scatter_wait(j - 1, y)
                if r == 0:
                    gather_start(j + 2, y)
                else:
                    @pl.when(t < (CPB - 1) // 3 - 1)
                    def _():
                        gather_start(j + 2, y)
                scatter_start(j, x)

        pl.loop(0, (CPB - 1) // 3)(triple)
        scatter_wait(CPB - 1, (CPB - 1) % 3)

    pl.loop(0, NB)(block)
    plsc.subcore_barrier()
    pltpu.sync_copy(
        agg.at[pl.ds(s * RPT, RPT)], out.at[c].at[pl.ds(s * RPT, RPT)]
    )


_sc_call = pl.kernel(
    _sc_body,
    out_type=jax.ShapeDtypeStruct((NC, NPAD, D), jnp.float32),
    mesh=plsc.VectorSubcoreMesh(core_axis_name="c", subcore_axis_name="s"),
    scratch_types=[
        pltpu.VMEM((CPB, C), jnp.int32),
        pltpu.VMEM((CPB, C), jnp.int32),
        pltpu.VMEM((CPB, C), jnp.float32),
        pltpu.VMEM((C, D), jnp.float32),
        pltpu.VMEM((C, D), jnp.float32),
        pltpu.VMEM((C, D), jnp.float32),
        pltpu.VMEM_SHARED((NPAD, D), jnp.float32),
        pltpu.SemaphoreType.DMA,
        pltpu.SemaphoreType.DMA,
        pltpu.SemaphoreType.DMA,
        pltpu.SemaphoreType.DMA,
        pltpu.SemaphoreType.DMA,
        pltpu.SemaphoreType.DMA,
    ],
    compiler_params=pltpu.CompilerParams(needs_layout_passes=False),
)


def kernel(H, edge_index, norm_factor, W_node, b_node, W_update, b_update):
    RB = 1000
    henc = pl.pallas_call(
        _enc_body,
        grid=(N // RB,),
        in_specs=[
            pl.BlockSpec((RB, D), lambda i: (i, 0)),
            pl.BlockSpec((D, D), lambda i: (0, 0)),
            pl.BlockSpec((1, D), lambda i: (0, 0)),
        ],
        out_specs=pl.BlockSpec((RB, D), lambda i: (i, 0)),
        out_shape=jax.ShapeDtypeStruct((N, D), jnp.float32),
    )(H, W_node, b_node.reshape(1, D))

    u3 = edge_index[0].astype(jnp.int32).reshape(NW, NB, CPB, C)
    v3 = edge_index[1].astype(jnp.int32).reshape(NW, NB, CPB, C)
    n3 = norm_factor.astype(jnp.float32).reshape(NW, NB, CPB, C)

    partials = _sc_call(henc, u3, v3, n3)

    out = pl.pallas_call(
        _update_body,
        grid=(N // RB,),
        in_specs=[
            pl.BlockSpec((NC, RB, D), lambda i: (0, i, 0)),
            pl.BlockSpec((RB, D), lambda i: (i, 0)),
            pl.BlockSpec((D, D), lambda i: (0, 0)),
            pl.BlockSpec((1, D), lambda i: (0, 0)),
        ],
        out_specs=pl.BlockSpec((RB, D), lambda i: (i, 0)),
        out_shape=jax.ShapeDtypeStruct((N, D), jnp.float32),
    )(partials, henc, W_update, b_update.reshape(1, D))
    return out
